# baseline (speedup 1.0000x reference)
import jax
import jax.numpy as jnp
from jax.experimental import pallas as pl
from jax.experimental.pallas import tpu as pltpu

_BM = 512
_BK = 2048


def _fused_kernel(b_ref, w_ref, bias_ref, a_ref, outt_ref, h_ref):
    i = pl.program_id(0)
    j = pl.program_id(1)

    @pl.when((i == 0) & (j == 0))
    def _():
        h_ref[...] = (
            jnp.dot(b_ref[...], w_ref[...], preferred_element_type=jnp.float32)
            + bias_ref[...]
        )

    partial = jax.lax.dot_general(
        h_ref[pl.ds(j * _BK, _BK), :],
        a_ref[...],
        dimension_numbers=(((0,), (1,)), ((), ())),
        preferred_element_type=jnp.float32,
    )

    @pl.when(j == 0)
    def _():
        outt_ref[...] = partial

    @pl.when(j != 0)
    def _():
        outt_ref[...] += partial


def kernel(a, b, W, bias):
    n, k = a.shape
    d_in = b.shape[1]
    d_out = W.shape[1]
    bias2d = bias.reshape(1, d_out)

    outt = pl.pallas_call(
        _fused_kernel,
        grid=(n // _BM, k // _BK),
        in_specs=[
            pl.BlockSpec((k, d_in), lambda i, j: (0, 0)),
            pl.BlockSpec((d_in, d_out), lambda i, j: (0, 0)),
            pl.BlockSpec((1, d_out), lambda i, j: (0, 0)),
            pl.BlockSpec((_BM, _BK), lambda i, j: (i, j)),
        ],
        out_specs=pl.BlockSpec((d_out, _BM), lambda i, j: (0, i)),
        out_shape=jax.ShapeDtypeStruct((d_out, n), jnp.float32),
        scratch_shapes=[pltpu.VMEM((k, d_out), jnp.float32)],
        compiler_params=pltpu.CompilerParams(
            dimension_semantics=("arbitrary", "arbitrary"),
        ),
    )(b, W, bias2d, a)
    return outt.T


# speedup vs baseline: 1.1059x; 1.1059x over previous
import jax
import jax.numpy as jnp
from jax.experimental import pallas as pl
from jax.experimental.pallas import tpu as pltpu

_BM = 512


def _fused_kernel(b_ref, w_ref, bias_ref, a_ref, outt_ref, h_ref):
    @pl.when(pl.program_id(0) == 0)
    def _():
        h_ref[...] = (
            jnp.dot(b_ref[...], w_ref[...], preferred_element_type=jnp.float32)
            + bias_ref[...]
        )

    outt_ref[...] = jax.lax.dot_general(
        h_ref[...],
        a_ref[...],
        dimension_numbers=(((0,), (1,)), ((), ())),
        preferred_element_type=jnp.float32,
    )


def kernel(a, b, W, bias):
    n, k = a.shape
    d_in = b.shape[1]
    d_out = W.shape[1]
    bias2d = bias.reshape(1, d_out)

    outt = pl.pallas_call(
        _fused_kernel,
        grid=(n // _BM,),
        in_specs=[
            pl.BlockSpec((k, d_in), lambda i: (0, 0)),
            pl.BlockSpec((d_in, d_out), lambda i: (0, 0)),
            pl.BlockSpec((1, d_out), lambda i: (0, 0)),
            pl.BlockSpec((_BM, k), lambda i: (i, 0)),
        ],
        out_specs=pl.BlockSpec((d_out, _BM), lambda i: (0, i)),
        out_shape=jax.ShapeDtypeStruct((d_out, n), jnp.float32),
        scratch_shapes=[pltpu.VMEM((k, d_out), jnp.float32)],
        compiler_params=pltpu.CompilerParams(
            dimension_semantics=("arbitrary",),
            fuse_transposed_lhs_in_matmul=True,
        ),
    )(b, W, bias2d, a)
    return outt.T


# speedup vs baseline: 1.1620x; 1.0507x over previous
import jax
import jax.numpy as jnp
from jax.experimental import pallas as pl
from jax.experimental.pallas import tpu as pltpu

_BM = 512


def _fused_kernel(b_ref, w_ref, bias_ref, a_ref, outt_ref, h_ref):
    @pl.when(pl.program_id(0) == 0)
    def _():
        h_ref[...] = (
            jnp.dot(b_ref[...], w_ref[...], preferred_element_type=jnp.float32)
            + bias_ref[...]
        )

    outt_ref[...] = jax.lax.dot_general(
        h_ref[...],
        a_ref[...],
        dimension_numbers=(((0,), (1,)), ((), ())),
        preferred_element_type=jnp.float32,
    )


def kernel(a, b, W, bias):
    n, k = a.shape
    d_in = b.shape[1]
    d_out = W.shape[1]
    bias2d = bias.reshape(1, d_out)

    outt = pl.pallas_call(
        _fused_kernel,
        grid=(n // _BM,),
        in_specs=[
            pl.BlockSpec((k, d_in), lambda i: (0, 0)),
            pl.BlockSpec((d_in, d_out), lambda i: (0, 0)),
            pl.BlockSpec((1, d_out), lambda i: (0, 0)),
            pl.BlockSpec((_BM, k), lambda i: (i, 0)),
        ],
        out_specs=pl.BlockSpec((d_out, _BM), lambda i: (0, i)),
        out_shape=jax.ShapeDtypeStruct((d_out, n), jnp.float32),
        scratch_shapes=[pltpu.VMEM((k, d_out), jnp.float32)],
        compiler_params=pltpu.CompilerParams(
            dimension_semantics=("arbitrary",),
        ),
    )(b, W, bias2d, a)
    return outt.T
